# baseline (speedup 1.0000x reference)
import functools

import jax
import jax.numpy as jnp
from jax.experimental import pallas as pl
from jax.experimental.pallas import tpu as pltpu

KNN = 16
QB = 128
CB = 512
IMAX = jnp.iinfo(jnp.int32).max
INF = float("inf")


def _knn_kernel(nk_real, qt_ref, k_ref, dv_ref, di_ref, s_s, bv_s, bi_s):
    ci = pl.program_id(0)
    qi = pl.program_id(1)
    ncb = pl.num_programs(0)
    cb = k_ref.shape[0]
    qrow = pl.ds(qi * KNN, KNN)

    @pl.when(ci == 0)
    def _init():
        bv_s[qrow, :] = jnp.full((KNN, QB), INF, jnp.float32)
        bi_s[qrow, :] = jnp.full((KNN, QB), IMAX, jnp.int32)

    k = k_ref[...]
    qt = qt_ref[...]
    dims = (((1,), (0,)), ((), ()))
    dotm2 = jax.lax.dot_general(k, qt * -2.0, dims,
                                precision=jax.lax.Precision.HIGHEST,
                                preferred_element_type=jnp.float32)
    ksq_b = jax.lax.dot_general(k * k, jnp.ones(qt.shape, jnp.float32), dims,
                                precision=jax.lax.Precision.HIGHEST,
                                preferred_element_type=jnp.float32)
    s = ksq_b + dotm2
    base = ci * cb
    if nk_real is not None:
        rows_always = jax.lax.broadcasted_iota(jnp.int32, (cb, QB), 0) + base
        s = jnp.where(rows_always < nk_real, s, INF)
    s_s[...] = s
    m = jnp.min(s, axis=0)

    wv_cur = jnp.max(bv_s[qrow, :], axis=0)
    need = jnp.any(m <= wv_cur)

    @pl.when(need)
    def _merge():
        sv = s_s[...]
        rows = jax.lax.broadcasted_iota(jnp.int32, (cb, QB), 0) + base
        bv0 = bv_s[qrow, :]
        bi0 = bi_s[qrow, :]
        srows = jax.lax.broadcasted_iota(jnp.int32, (KNN, QB), 0)

        def ins_mask(mv, mi, bv, bi):
            wv = jnp.max(bv, axis=0)
            wi = jnp.max(jnp.where(bv == wv[None, :], bi, -1), axis=0)
            ins = (mv < wv) | ((mv == wv) & (mi < wi) & (mv < INF))
            return ins, wv, wi

        im0 = jnp.min(jnp.where(sv == m[None, :], rows, IMAX), axis=0)
        ins0, _, _ = ins_mask(m, im0, bv0, bi0)

        def cond(c):
            return c[4]

        def body(c):
            mv, mi, bv, bi, _ = c
            ins, wv, wi = ins_mask(mv, mi, bv, bi)
            match = (bv == wv[None, :]) & (bi == wi[None, :])
            wrow = jnp.min(jnp.where(match, srows, KNN), axis=0)
            sel = (srows == wrow[None, :]) & ins[None, :]
            bv = jnp.where(sel, mv[None, :], bv)
            bi = jnp.where(sel, mi[None, :], bi)
            keep = (sv > mv[None, :]) | ((sv == mv[None, :]) & (rows > mi[None, :]))
            vm = jnp.where(keep, sv, INF)
            mv2 = jnp.min(vm, axis=0)
            mi2 = jnp.min(jnp.where(vm == mv2[None, :], rows, IMAX), axis=0)
            ins2, _, _ = ins_mask(mv2, mi2, bv, bi)
            return (mv2, mi2, bv, bi, jnp.any(ins2))

        _, _, bvF, biF, _ = jax.lax.while_loop(
            cond, body, (m, im0, bv0, bi0, jnp.any(ins0)))
        bv_s[qrow, :] = bvF
        bi_s[qrow, :] = biF

    @pl.when(ci == ncb - 1)
    def _emit():
        qsq = jnp.sum(qt * qt, axis=0)
        bv = bv_s[qrow, :]
        bi = bi_s[qrow, :]
        fv = jnp.full((QB,), -INF, jnp.float32)
        fi = jnp.full((QB,), -1, jnp.int32)
        for r in range(KNN):
            keep = (bv > fv[None, :]) | ((bv == fv[None, :]) & (bi > fi[None, :]))
            vm = jnp.where(keep, bv, INF)
            mv = jnp.min(vm, axis=0)
            mi = jnp.min(jnp.where(vm == mv[None, :], bi, IMAX), axis=0)
            dv_ref[r, :] = jnp.maximum(qsq + mv, 0.0)
            di_ref[r, :] = mi
            fv, fi = mv, mi


def kernel(queries, keys):
    nq, d = queries.shape
    nk, _ = keys.shape

    nq_pad = -(-nq // QB) * QB
    if nq_pad != nq:
        queries = jnp.pad(queries, ((0, nq_pad - nq), (0, 0)))
    ncb = -(-nk // CB)
    nk_pad = ncb * CB
    nk_real = None
    if nk_pad != nk:
        keys = jnp.pad(keys, ((0, nk_pad - nk), (0, 0)))
        nk_real = nk

    qt = queries.T
    nqb = nq_pad // QB
    dv, di = pl.pallas_call(
        functools.partial(_knn_kernel, nk_real),
        grid=(ncb, nqb),
        in_specs=[
            pl.BlockSpec((d, QB), lambda i, j: (0, j)),
            pl.BlockSpec((CB, d), lambda i, j: (i, 0)),
        ],
        out_specs=[
            pl.BlockSpec((KNN, QB), lambda i, j: (0, j)),
            pl.BlockSpec((KNN, QB), lambda i, j: (0, j)),
        ],
        out_shape=[
            jax.ShapeDtypeStruct((KNN, nq_pad), jnp.float32),
            jax.ShapeDtypeStruct((KNN, nq_pad), jnp.int32),
        ],
        scratch_shapes=[
            pltpu.VMEM((CB, QB), jnp.float32),
            pltpu.VMEM((nqb * KNN, QB), jnp.float32),
            pltpu.VMEM((nqb * KNN, QB), jnp.int32),
        ],
        compiler_params=pltpu.CompilerParams(
            dimension_semantics=("arbitrary", "arbitrary"),
        ),
    )(qt, keys)
    return dv.T[:nq], di.T[:nq]


# speedup vs baseline: 1.1109x; 1.1109x over previous
import functools

import jax
import jax.numpy as jnp
from jax.experimental import pallas as pl
from jax.experimental.pallas import tpu as pltpu

KNN = 16
QB = 128
CB = 512
IMAX = jnp.iinfo(jnp.int32).max
INF = float("inf")


def _knn_kernel(nk_real, qt_ref, k_ref, dv_ref, di_ref, s_s, bv_s, bi_s):
    ci = pl.program_id(0)
    qi = pl.program_id(1)
    ncb = pl.num_programs(0)
    cb = k_ref.shape[0]
    qrow = pl.ds(qi * KNN, KNN)

    @pl.when(ci == 0)
    def _init():
        bv_s[qrow, :] = jnp.full((KNN, QB), INF, jnp.float32)
        bi_s[qrow, :] = jnp.full((KNN, QB), IMAX, jnp.int32)

    k = k_ref[...]
    qt = qt_ref[...]
    dims = (((1,), (0,)), ((), ()))
    dotm2 = jax.lax.dot_general(k.astype(jnp.bfloat16),
                                (qt * -2.0).astype(jnp.bfloat16), dims,
                                preferred_element_type=jnp.float32)
    ksq_b = jax.lax.dot_general(k * k, jnp.ones(qt.shape, jnp.float32), dims,
                                precision=jax.lax.Precision.HIGHEST,
                                preferred_element_type=jnp.float32)
    s = ksq_b + dotm2
    base = ci * cb
    if nk_real is not None:
        rows_always = jax.lax.broadcasted_iota(jnp.int32, (cb, QB), 0) + base
        s = jnp.where(rows_always < nk_real, s, INF)
    s_s[...] = s
    m = jnp.min(s, axis=0)

    wv_cur = jnp.max(bv_s[qrow, :], axis=0)
    need = jnp.any(m <= wv_cur)

    @pl.when(need)
    def _merge():
        sv = s_s[...]
        rows = jax.lax.broadcasted_iota(jnp.int32, (cb, QB), 0) + base
        bv0 = bv_s[qrow, :]
        bi0 = bi_s[qrow, :]
        srows = jax.lax.broadcasted_iota(jnp.int32, (KNN, QB), 0)

        def ins_mask(mv, mi, bv, bi):
            wv = jnp.max(bv, axis=0)
            wi = jnp.max(jnp.where(bv == wv[None, :], bi, -1), axis=0)
            ins = (mv < wv) | ((mv == wv) & (mi < wi) & (mv < INF))
            return ins, wv, wi

        im0 = jnp.min(jnp.where(sv == m[None, :], rows, IMAX), axis=0)
        ins0, _, _ = ins_mask(m, im0, bv0, bi0)

        def cond(c):
            return c[4]

        def body(c):
            mv, mi, bv, bi, _ = c
            ins, wv, wi = ins_mask(mv, mi, bv, bi)
            match = (bv == wv[None, :]) & (bi == wi[None, :])
            wrow = jnp.min(jnp.where(match, srows, KNN), axis=0)
            sel = (srows == wrow[None, :]) & ins[None, :]
            bv = jnp.where(sel, mv[None, :], bv)
            bi = jnp.where(sel, mi[None, :], bi)
            keep = (sv > mv[None, :]) | ((sv == mv[None, :]) & (rows > mi[None, :]))
            vm = jnp.where(keep, sv, INF)
            mv2 = jnp.min(vm, axis=0)
            mi2 = jnp.min(jnp.where(vm == mv2[None, :], rows, IMAX), axis=0)
            ins2, _, _ = ins_mask(mv2, mi2, bv, bi)
            return (mv2, mi2, bv, bi, jnp.any(ins2))

        _, _, bvF, biF, _ = jax.lax.while_loop(
            cond, body, (m, im0, bv0, bi0, jnp.any(ins0)))
        bv_s[qrow, :] = bvF
        bi_s[qrow, :] = biF

    @pl.when(ci == ncb - 1)
    def _emit():
        qsq = jnp.sum(qt * qt, axis=0)
        bv = bv_s[qrow, :]
        bi = bi_s[qrow, :]
        fv = jnp.full((QB,), -INF, jnp.float32)
        fi = jnp.full((QB,), -1, jnp.int32)
        for r in range(KNN):
            keep = (bv > fv[None, :]) | ((bv == fv[None, :]) & (bi > fi[None, :]))
            vm = jnp.where(keep, bv, INF)
            mv = jnp.min(vm, axis=0)
            mi = jnp.min(jnp.where(vm == mv[None, :], bi, IMAX), axis=0)
            dv_ref[r, :] = jnp.maximum(qsq + mv, 0.0)
            di_ref[r, :] = mi
            fv, fi = mv, mi


def kernel(queries, keys):
    nq, d = queries.shape
    nk, _ = keys.shape

    nq_pad = -(-nq // QB) * QB
    if nq_pad != nq:
        queries = jnp.pad(queries, ((0, nq_pad - nq), (0, 0)))
    ncb = -(-nk // CB)
    nk_pad = ncb * CB
    nk_real = None
    if nk_pad != nk:
        keys = jnp.pad(keys, ((0, nk_pad - nk), (0, 0)))
        nk_real = nk

    qt = queries.T
    nqb = nq_pad // QB
    dv, di = pl.pallas_call(
        functools.partial(_knn_kernel, nk_real),
        grid=(ncb, nqb),
        in_specs=[
            pl.BlockSpec((d, QB), lambda i, j: (0, j)),
            pl.BlockSpec((CB, d), lambda i, j: (i, 0)),
        ],
        out_specs=[
            pl.BlockSpec((KNN, QB), lambda i, j: (0, j)),
            pl.BlockSpec((KNN, QB), lambda i, j: (0, j)),
        ],
        out_shape=[
            jax.ShapeDtypeStruct((KNN, nq_pad), jnp.float32),
            jax.ShapeDtypeStruct((KNN, nq_pad), jnp.int32),
        ],
        scratch_shapes=[
            pltpu.VMEM((CB, QB), jnp.float32),
            pltpu.VMEM((nqb * KNN, QB), jnp.float32),
            pltpu.VMEM((nqb * KNN, QB), jnp.int32),
        ],
        compiler_params=pltpu.CompilerParams(
            dimension_semantics=("arbitrary", "arbitrary"),
        ),
    )(qt, keys)
    return dv.T[:nq], di.T[:nq]


# speedup vs baseline: 1.4117x; 1.2707x over previous
import functools

import jax
import jax.numpy as jnp
from jax.experimental import pallas as pl
from jax.experimental.pallas import tpu as pltpu

KNN = 16
QB = 128
CB = 512
IMAX = jnp.iinfo(jnp.int32).max
INF = float("inf")


def _knn_kernel(nk_real, qt_ref, k_ref, dv_ref, di_ref, ksqb_s, s_s, bv_s, bi_s):
    ci = pl.program_id(0)
    qi = pl.program_id(1)
    ncb = pl.num_programs(0)
    cb = k_ref.shape[0]
    qrow = pl.ds(qi * KNN, KNN)

    @pl.when(ci == 0)
    def _init():
        bv_s[qrow, :] = jnp.full((KNN, QB), INF, jnp.float32)
        bi_s[qrow, :] = jnp.full((KNN, QB), IMAX, jnp.int32)

    k = k_ref[...]
    qt = qt_ref[...]
    dims = (((1,), (0,)), ((), ()))

    @pl.when(qi == 0)
    def _ksq():
        ksq_col = jnp.sum(k * k, axis=1, keepdims=True)
        ksqb_s[...] = jax.lax.dot_general(
            ksq_col, jnp.ones((1, QB), jnp.float32), dims,
            precision=jax.lax.Precision.HIGHEST,
            preferred_element_type=jnp.float32)

    dotm2 = jax.lax.dot_general(k.astype(jnp.bfloat16),
                                (qt * -2.0).astype(jnp.bfloat16), dims,
                                preferred_element_type=jnp.float32)
    s = ksqb_s[...] + dotm2
    base = ci * cb
    if nk_real is not None:
        rows_always = jax.lax.broadcasted_iota(jnp.int32, (cb, QB), 0) + base
        s = jnp.where(rows_always < nk_real, s, INF)
    s_s[...] = s
    m = jnp.min(s, axis=0)

    def _merge():
        sv = s_s[...]
        rows = jax.lax.broadcasted_iota(jnp.int32, (cb, QB), 0) + base
        bv0 = bv_s[qrow, :]
        bi0 = bi_s[qrow, :]
        srows = jax.lax.broadcasted_iota(jnp.int32, (KNN, QB), 0)

        def ins_mask(mv, mi, bv, bi):
            wv = jnp.max(bv, axis=0)
            wi = jnp.max(jnp.where(bv == wv[None, :], bi, -1), axis=0)
            ins = (mv < wv) | ((mv == wv) & (mi < wi) & (mv < INF))
            return ins, wv, wi

        im0 = jnp.min(jnp.where(sv == m[None, :], rows, IMAX), axis=0)
        ins0, _, _ = ins_mask(m, im0, bv0, bi0)

        def cond(c):
            return c[4]

        def body(c):
            mv, mi, bv, bi, _ = c
            ins, wv, wi = ins_mask(mv, mi, bv, bi)
            match = (bv == wv[None, :]) & (bi == wi[None, :])
            wrow = jnp.min(jnp.where(match, srows, KNN), axis=0)
            sel = (srows == wrow[None, :]) & ins[None, :]
            bv = jnp.where(sel, mv[None, :], bv)
            bi = jnp.where(sel, mi[None, :], bi)
            keep = (sv > mv[None, :]) | ((sv == mv[None, :]) & (rows > mi[None, :]))
            vm = jnp.where(keep, sv, INF)
            mv2 = jnp.min(vm, axis=0)
            mi2 = jnp.min(jnp.where(vm == mv2[None, :], rows, IMAX), axis=0)
            ins2, _, _ = ins_mask(mv2, mi2, bv, bi)
            return (mv2, mi2, bv, bi, jnp.any(ins2))

        _, _, bvF, biF, _ = jax.lax.while_loop(
            cond, body, (m, im0, bv0, bi0, jnp.any(ins0)))
        bv_s[qrow, :] = bvF
        bi_s[qrow, :] = biF

    _merge()

    @pl.when(ci == ncb - 1)
    def _emit():
        qsq = jnp.sum(qt * qt, axis=0)
        bv = bv_s[qrow, :]
        bi = bi_s[qrow, :]
        fv = jnp.full((QB,), -INF, jnp.float32)
        fi = jnp.full((QB,), -1, jnp.int32)
        for r in range(KNN):
            keep = (bv > fv[None, :]) | ((bv == fv[None, :]) & (bi > fi[None, :]))
            vm = jnp.where(keep, bv, INF)
            mv = jnp.min(vm, axis=0)
            mi = jnp.min(jnp.where(vm == mv[None, :], bi, IMAX), axis=0)
            dv_ref[r, :] = jnp.maximum(qsq + mv, 0.0)
            di_ref[r, :] = mi
            fv, fi = mv, mi


def kernel(queries, keys):
    nq, d = queries.shape
    nk, _ = keys.shape

    nq_pad = -(-nq // QB) * QB
    if nq_pad != nq:
        queries = jnp.pad(queries, ((0, nq_pad - nq), (0, 0)))
    ncb = -(-nk // CB)
    nk_pad = ncb * CB
    nk_real = None
    if nk_pad != nk:
        keys = jnp.pad(keys, ((0, nk_pad - nk), (0, 0)))
        nk_real = nk

    qt = queries.T
    nqb = nq_pad // QB
    dv, di = pl.pallas_call(
        functools.partial(_knn_kernel, nk_real),
        grid=(ncb, nqb),
        in_specs=[
            pl.BlockSpec((d, QB), lambda i, j: (0, j)),
            pl.BlockSpec((CB, d), lambda i, j: (i, 0)),
        ],
        out_specs=[
            pl.BlockSpec((KNN, QB), lambda i, j: (0, j)),
            pl.BlockSpec((KNN, QB), lambda i, j: (0, j)),
        ],
        out_shape=[
            jax.ShapeDtypeStruct((KNN, nq_pad), jnp.float32),
            jax.ShapeDtypeStruct((KNN, nq_pad), jnp.int32),
        ],
        scratch_shapes=[
            pltpu.VMEM((CB, QB), jnp.float32),
            pltpu.VMEM((CB, QB), jnp.float32),
            pltpu.VMEM((nqb * KNN, QB), jnp.float32),
            pltpu.VMEM((nqb * KNN, QB), jnp.int32),
        ],
        compiler_params=pltpu.CompilerParams(
            dimension_semantics=("arbitrary", "arbitrary"),
        ),
    )(qt, keys)
    return dv.T[:nq], di.T[:nq]


# speedup vs baseline: 1.7822x; 1.2625x over previous
import functools

import jax
import jax.numpy as jnp
from jax.experimental import pallas as pl
from jax.experimental.pallas import tpu as pltpu

KNN = 16
QB = 128
CB = 1024
IMAX = jnp.iinfo(jnp.int32).max
INF = float("inf")


def _knn_kernel(nk_real, qt_ref, k_ref, dv_ref, di_ref, ksqb_s, s_s, bv_s, bi_s):
    ci = pl.program_id(0)
    qi = pl.program_id(1)
    ncb = pl.num_programs(0)
    cb = k_ref.shape[0]
    qrow = pl.ds(qi * KNN, KNN)

    @pl.when(ci == 0)
    def _init():
        bv_s[qrow, :] = jnp.full((KNN, QB), INF, jnp.float32)
        bi_s[qrow, :] = jnp.full((KNN, QB), IMAX, jnp.int32)

    k = k_ref[...]
    qt = qt_ref[...]
    dims = (((1,), (0,)), ((), ()))

    @pl.when(qi == 0)
    def _ksq():
        ksq_col = jnp.sum(k * k, axis=1, keepdims=True)
        ksqb_s[...] = jax.lax.dot_general(
            ksq_col, jnp.ones((1, QB), jnp.float32), dims,
            precision=jax.lax.Precision.HIGHEST,
            preferred_element_type=jnp.float32)

    dotm2 = jax.lax.dot_general(k.astype(jnp.bfloat16),
                                (qt * -2.0).astype(jnp.bfloat16), dims,
                                preferred_element_type=jnp.float32)
    s = ksqb_s[...] + dotm2
    base = ci * cb
    if nk_real is not None:
        rows_always = jax.lax.broadcasted_iota(jnp.int32, (cb, QB), 0) + base
        s = jnp.where(rows_always < nk_real, s, INF)
    s_s[...] = s
    m = jnp.min(s, axis=0)

    def _merge():
        sv = s_s[...]
        rows = jax.lax.broadcasted_iota(jnp.int32, (cb, QB), 0) + base
        bv0 = bv_s[qrow, :]
        bi0 = bi_s[qrow, :]
        srows = jax.lax.broadcasted_iota(jnp.int32, (KNN, QB), 0)

        def ins_mask(mv, mi, bv, bi):
            wv = jnp.max(bv, axis=0)
            wi = jnp.max(jnp.where(bv == wv[None, :], bi, -1), axis=0)
            ins = (mv < wv) | ((mv == wv) & (mi < wi) & (mv < INF))
            return ins, wv, wi

        im0 = jnp.min(jnp.where(sv == m[None, :], rows, IMAX), axis=0)
        ins0, _, _ = ins_mask(m, im0, bv0, bi0)

        def cond(c):
            return c[4]

        def body(c):
            mv, mi, bv, bi, _ = c
            ins, wv, wi = ins_mask(mv, mi, bv, bi)
            match = (bv == wv[None, :]) & (bi == wi[None, :])
            wrow = jnp.min(jnp.where(match, srows, KNN), axis=0)
            sel = (srows == wrow[None, :]) & ins[None, :]
            bv = jnp.where(sel, mv[None, :], bv)
            bi = jnp.where(sel, mi[None, :], bi)
            keep = (sv > mv[None, :]) | ((sv == mv[None, :]) & (rows > mi[None, :]))
            vm = jnp.where(keep, sv, INF)
            mv2 = jnp.min(vm, axis=0)
            mi2 = jnp.min(jnp.where(vm == mv2[None, :], rows, IMAX), axis=0)
            ins2, _, _ = ins_mask(mv2, mi2, bv, bi)
            return (mv2, mi2, bv, bi, jnp.any(ins2))

        _, _, bvF, biF, _ = jax.lax.while_loop(
            cond, body, (m, im0, bv0, bi0, jnp.any(ins0)))
        bv_s[qrow, :] = bvF
        bi_s[qrow, :] = biF

    _merge()

    @pl.when(ci == ncb - 1)
    def _emit():
        qsq = jnp.sum(qt * qt, axis=0)
        bv = bv_s[qrow, :]
        bi = bi_s[qrow, :]
        fv = jnp.full((QB,), -INF, jnp.float32)
        fi = jnp.full((QB,), -1, jnp.int32)
        for r in range(KNN):
            keep = (bv > fv[None, :]) | ((bv == fv[None, :]) & (bi > fi[None, :]))
            vm = jnp.where(keep, bv, INF)
            mv = jnp.min(vm, axis=0)
            mi = jnp.min(jnp.where(vm == mv[None, :], bi, IMAX), axis=0)
            dv_ref[r, :] = jnp.maximum(qsq + mv, 0.0)
            di_ref[r, :] = mi
            fv, fi = mv, mi


def kernel(queries, keys):
    nq, d = queries.shape
    nk, _ = keys.shape

    nq_pad = -(-nq // QB) * QB
    if nq_pad != nq:
        queries = jnp.pad(queries, ((0, nq_pad - nq), (0, 0)))
    ncb = -(-nk // CB)
    nk_pad = ncb * CB
    nk_real = None
    if nk_pad != nk:
        keys = jnp.pad(keys, ((0, nk_pad - nk), (0, 0)))
        nk_real = nk

    qt = queries.T
    nqb = nq_pad // QB
    dv, di = pl.pallas_call(
        functools.partial(_knn_kernel, nk_real),
        grid=(ncb, nqb),
        in_specs=[
            pl.BlockSpec((d, QB), lambda i, j: (0, j)),
            pl.BlockSpec((CB, d), lambda i, j: (i, 0)),
        ],
        out_specs=[
            pl.BlockSpec((KNN, QB), lambda i, j: (0, j)),
            pl.BlockSpec((KNN, QB), lambda i, j: (0, j)),
        ],
        out_shape=[
            jax.ShapeDtypeStruct((KNN, nq_pad), jnp.float32),
            jax.ShapeDtypeStruct((KNN, nq_pad), jnp.int32),
        ],
        scratch_shapes=[
            pltpu.VMEM((CB, QB), jnp.float32),
            pltpu.VMEM((CB, QB), jnp.float32),
            pltpu.VMEM((nqb * KNN, QB), jnp.float32),
            pltpu.VMEM((nqb * KNN, QB), jnp.int32),
        ],
        compiler_params=pltpu.CompilerParams(
            dimension_semantics=("arbitrary", "arbitrary"),
        ),
    )(qt, keys)
    return dv.T[:nq], di.T[:nq]
